# Initial kernel scaffold; baseline (speedup 1.0000x reference)
import jax
import jax.numpy as jnp
from jax.experimental import pallas as pl


def kernel(x, router, w_up_gate, w_down):
    raise NotImplementedError("write your pallas kernel here")


# speedup vs baseline: 1.4362x; 1.4362x over previous
import functools

import jax
import jax.numpy as jnp
from jax.experimental import pallas as pl
from jax.experimental.pallas import tpu as pltpu

H = 1024
I = 2816
E = 8
K = 2
B, S = 2, 2048
T = B * S
A = K * T

TB = 512
NTB = T // TB

TILE = 256
NT = A // TILE + E
PAD = NT * TILE
NJ = 2
IC = I // NJ


def _router_body(x_ref, r_ref, i1_ref, i2_ref, w1_ref, w2_ref, r0_ref, r1_ref,
                 cnt_ref, stats_ref, carry, psum, zsq):
    bi = pl.program_id(0)

    @pl.when(bi == 0)
    def _init():
        carry[...] = jnp.zeros_like(carry)
        psum[...] = jnp.zeros_like(psum)
        zsq[0, 0] = 0.0

    x = x_ref[...]
    logits = jnp.dot(x, r_ref[...], preferred_element_type=jnp.float32)

    lanes = jax.lax.broadcasted_iota(jnp.int32, (TB, E), 1)
    m1 = jnp.max(logits, axis=-1, keepdims=True)
    i1 = jnp.min(jnp.where(logits == m1, lanes, E), axis=-1, keepdims=True)
    oh1 = (lanes == i1).astype(jnp.float32)
    l2 = jnp.where(lanes == i1, jnp.float32(-1e30), logits)
    m2 = jnp.max(l2, axis=-1, keepdims=True)
    i2 = jnp.min(jnp.where(l2 == m2, lanes, E), axis=-1, keepdims=True)
    oh2 = (lanes == i2).astype(jnp.float32)

    e2 = jnp.exp(m2 - m1)
    w1 = 1.0 / (1.0 + e2)
    w2 = e2 / (1.0 + e2)

    ex = jnp.exp(logits - m1)
    sex = jnp.sum(ex, axis=-1, keepdims=True)
    probs = ex / sex
    z = m1 + jnp.log(sex)

    rows = jax.lax.broadcasted_iota(jnp.int32, (TB, TB), 0)
    cols = jax.lax.broadcasted_iota(jnp.int32, (TB, TB), 1)
    tril = (cols < rows).astype(jnp.float32)
    ohb = oh1 + oh2
    base = jnp.dot(tril, ohb, preferred_element_type=jnp.float32) + carry[...]
    r0 = jnp.sum(base * oh1, axis=-1, keepdims=True)
    r1 = jnp.sum(base * oh2, axis=-1, keepdims=True)

    carry[...] = carry[...] + jnp.sum(ohb, axis=0, keepdims=True)
    psum[...] = psum[...] + jnp.sum(probs, axis=0, keepdims=True)
    zsq[0, 0] = zsq[0, 0] + jnp.sum(z * z)

    i1_ref[...] = i1
    i2_ref[...] = i2
    w1_ref[...] = w1
    w2_ref[...] = w2
    r0_ref[...] = r0
    r1_ref[...] = r1

    @pl.when(bi == NTB - 1)
    def _fin():
        cnt = carry[...]
        cnt_ref[...] = cnt
        af = cnt / jnp.float32(A)
        ent = -jnp.sum(af * jnp.log(af + 1e-6))
        pm = psum[...] / jnp.float32(T)
        lb = jnp.float32(E) * jnp.sum(af * jnp.float32(K) * pm)
        stats_ref[0, 0] = ent
        stats_ref[0, 1] = lb
        stats_ref[0, 2] = zsq[0, 0] / jnp.float32(T)


def _router_call(x_flat, router):
    f32 = jnp.float32
    outs = (
        jax.ShapeDtypeStruct((T, 1), jnp.int32),
        jax.ShapeDtypeStruct((T, 1), jnp.int32),
        jax.ShapeDtypeStruct((T, 1), f32),
        jax.ShapeDtypeStruct((T, 1), f32),
        jax.ShapeDtypeStruct((T, 1), f32),
        jax.ShapeDtypeStruct((T, 1), f32),
        jax.ShapeDtypeStruct((1, E), f32),
        jax.ShapeDtypeStruct((1, 8), f32),
    )
    col = pl.BlockSpec((TB, 1), lambda i: (i, 0))
    return pl.pallas_call(
        _router_body,
        grid=(NTB,),
        in_specs=[
            pl.BlockSpec((TB, H), lambda i: (i, 0)),
            pl.BlockSpec((H, E), lambda i: (0, 0)),
        ],
        out_specs=(col, col, col, col, col, col,
                   pl.BlockSpec((1, E), lambda i: (0, 0)),
                   pl.BlockSpec((1, 8), lambda i: (0, 0),
                                memory_space=pltpu.SMEM)),
        out_shape=outs,
        scratch_shapes=[
            pltpu.VMEM((1, E), f32),
            pltpu.VMEM((1, E), f32),
            pltpu.SMEM((1, 1), f32),
        ],
    )(x_flat, router)


def _mlp_body(te_ref, act_ref, xs_ref, wg_ref, wu_ref, wd_ref, out_ref):
    i = pl.program_id(0)
    j = pl.program_id(1)

    @pl.when(act_ref[i] == 1)
    def _():
        xs = xs_ref[...]
        g = jnp.dot(xs, wg_ref[0], preferred_element_type=jnp.float32)
        u = jnp.dot(xs, wu_ref[0], preferred_element_type=jnp.float32)
        hh = g * jax.nn.sigmoid(g) * u
        y = jnp.dot(hh, wd_ref[0], preferred_element_type=jnp.float32)

        @pl.when(j == 0)
        def _():
            out_ref[...] = y

        @pl.when(j != 0)
        def _():
            out_ref[...] = out_ref[...] + y


def _snake(i, j):
    return jnp.where(i % 2 == 1, NJ - 1 - j, j)


def _mlp_call(te, act, xs, w_up_gate, w_down):
    grid_spec = pltpu.PrefetchScalarGridSpec(
        num_scalar_prefetch=2,
        grid=(NT, NJ),
        in_specs=[
            pl.BlockSpec((TILE, H), lambda i, j, te, act: (i, 0)),
            pl.BlockSpec((1, H, IC),
                         lambda i, j, te, act: (te[i], 0, _snake(i, j))),
            pl.BlockSpec((1, H, IC),
                         lambda i, j, te, act: (te[i], 0, NJ + _snake(i, j))),
            pl.BlockSpec((1, IC, H),
                         lambda i, j, te, act: (te[i], _snake(i, j), 0)),
        ],
        out_specs=pl.BlockSpec((TILE, H), lambda i, j, te, act: (i, 0)),
    )
    return pl.pallas_call(
        _mlp_body,
        grid_spec=grid_spec,
        out_shape=jax.ShapeDtypeStruct((PAD, H), jnp.float32),
    )(te, act, xs, w_up_gate, w_up_gate, w_down)


def _comb_body(g0_ref, g1_ref, w1_ref, w2_ref, o_ref):
    o_ref[...] = w1_ref[...] * g0_ref[...] + w2_ref[...] * g1_ref[...]


def _comb_call(g0, g1, w1, w2):
    row = pl.BlockSpec((TB, H), lambda i: (i, 0))
    col = pl.BlockSpec((TB, 1), lambda i: (i, 0))
    return pl.pallas_call(
        _comb_body,
        grid=(NTB,),
        in_specs=[row, row, col, col],
        out_specs=row,
        out_shape=jax.ShapeDtypeStruct((T, H), jnp.float32),
    )(g0, g1, w1, w2)


def kernel(x, router, w_up_gate, w_down):
    x_flat = x.reshape(T, H)
    i1, i2, w1, w2, r0, r1, cnt, stats = _router_call(x_flat, router)

    cnt_v = cnt[0]
    cnt_i = cnt_v.astype(jnp.int32)
    tiles_e = (cnt_i + TILE - 1) // TILE
    seg = jnp.concatenate([jnp.zeros(1, jnp.int32),
                           jnp.cumsum(tiles_e * TILE)[:-1]])
    pos0 = seg[i1[:, 0]] + r0[:, 0].astype(jnp.int32)
    pos1 = seg[i2[:, 0]] + r1[:, 0].astype(jnp.int32)
    tok = jnp.arange(T, dtype=jnp.int32)
    src = jnp.zeros(PAD, jnp.int32).at[pos0].set(tok).at[pos1].set(tok)

    cumt = jnp.cumsum(tiles_e)
    ti = jnp.arange(NT, dtype=jnp.int32)
    te = jnp.minimum(jnp.sum((ti[:, None] >= cumt[None, :]).astype(jnp.int32),
                             axis=1), E - 1).astype(jnp.int32)
    act = (ti < cumt[-1]).astype(jnp.int32)

    xs = jnp.take(x_flat, src, axis=0)
    ys = _mlp_call(te, act, xs, w_up_gate, w_down)
    g0 = jnp.take(ys, pos0, axis=0)
    g1 = jnp.take(ys, pos1, axis=0)
    routed = _comb_call(g0, g1, w1, w2).reshape(B, S, H)

    return (routed, cnt_v, stats[0, 0], stats[0, 1], stats[0, 2])
